# baseline (speedup 1.0000x reference)
import functools

import numpy as np
import jax
import jax.numpy as jnp
from jax import lax
from jax.experimental import pallas as pl
from jax.experimental.pallas import tpu as pltpu
from jax.experimental.pallas import tpu_sc as plsc

DM = 768
NB = 64
BATCH = 2
SEQ = 8192
CHUNK = SEQ // NB
TILE = 256
NT = SEQ // TILE
NC, NS = 2, 16
NW = NC * NS
RPW = SEQ // NW
SCH = 64


def _perm_body(buck_ref, out_ref):
    buck = buck_ref[0, 0, :]
    cid = lax.broadcasted_iota(jnp.int32, (NB, SEQ), 0)
    oh = jnp.where(cid == buck[None, :], 1.0, 0.0).astype(jnp.float32)

    r = lax.broadcasted_iota(jnp.int32, (TILE, TILE), 0)
    c = lax.broadcasted_iota(jnp.int32, (TILE, TILE), 1)
    tri = jnp.where(r <= c, 1.0, 0.0).astype(jnp.float32)

    lane_last = lax.broadcasted_iota(jnp.int32, (NB, TILE), 1) == (TILE - 1)
    run = jnp.zeros((NB, 1), jnp.float32)
    parts = []
    for t in range(NT):
        oh_t = oh[:, t * TILE:(t + 1) * TILE]
        c_t = lax.dot_general(oh_t, tri, (((1,), (0,)), ((), ())),
                              preferred_element_type=jnp.float32)
        c_t = c_t + run
        run = jnp.sum(jnp.where(lane_last, c_t, 0.0), axis=1, keepdims=True)
        parts.append(c_t)
    csum = jnp.concatenate(parts, axis=1)

    hi = jnp.floor(run * (1.0 / 32.0))
    lo = run - hi * 32.0
    rr = lax.broadcasted_iota(jnp.int32, (NB, NB), 0)
    cc = lax.broadcasted_iota(jnp.int32, (NB, NB), 1)
    lstrict = jnp.where(cc < rr, 1.0, 0.0).astype(jnp.float32)
    dn = (((1,), (0,)), ((), ()))
    starts = (lax.dot_general(lstrict, hi, dn, preferred_element_type=jnp.float32) * 32.0
              + lax.dot_general(lstrict, lo, dn, preferred_element_type=jnp.float32))

    val = oh * (csum + starts - 1.0)
    pos = jnp.sum(val, axis=0)
    out_ref[0, 0, :] = pos.astype(jnp.int32)


def _build_perm(buckets3):
    return pl.pallas_call(
        _perm_body,
        grid=(BATCH,),
        in_specs=[pl.BlockSpec((1, 1, SEQ), lambda i: (i, 0, 0))],
        out_specs=pl.BlockSpec((1, 1, SEQ), lambda i: (i, 0, 0)),
        out_shape=jax.ShapeDtypeStruct((BATCH, 1, SEQ), jnp.int32),
    )(buckets3)


def _sc_scatter_body(qf, kf, vf, didx, qg, kg, vg,
                     idx0, idx1, row0, row1,
                     isem0, isem1, gsem0, gsem1, ssem0, ssem1):
    cid = lax.axis_index("c")
    sid = lax.axis_index("s")
    base = (sid * NC + cid) * RPW
    srcs = (qf, kf, vf)
    dsts = (qg, kg, vg)
    idxb = (idx0, idx1)
    rowb = (row0, row1)
    isem = (isem0, isem1)
    gsem = (gsem0, gsem1)
    ssem = (ssem0, ssem1)
    nch = RPW // SCH
    ni = 3 * BATCH * nch

    def chunk(i):
        t, r = divmod(i, BATCH * nch)
        b, ch = divmod(r, nch)
        return t, b * SEQ + base + ch * SCH

    gdesc = {}

    def prefetch(i):
        s = i % 2
        t, row = chunk(i)
        d1 = pltpu.async_copy(didx.at[pl.ds(row, SCH)], idxb[s], isem[s])
        d2 = pltpu.async_copy(srcs[t].at[pl.ds(row, SCH)], rowb[s], gsem[s])
        gdesc[i] = (d1, d2)

    sdesc = {}
    prefetch(0)
    for i in range(ni):
        s = i % 2
        t, _ = chunk(i)
        d1, d2 = gdesc.pop(i)
        d1.wait()
        d2.wait()
        sdesc[i] = pltpu.async_copy(rowb[s], dsts[t].at[idxb[s]], ssem[s])
        if i + 1 < ni:
            if i >= 1:
                sdesc.pop(i - 1).wait()
            prefetch(i + 1)
    sdesc.pop(ni - 2).wait()
    sdesc.pop(ni - 1).wait()


@functools.cache
def _sc_scatter():
    rows = jax.ShapeDtypeStruct((BATCH * SEQ, DM), jnp.float32)
    mesh = plsc.VectorSubcoreMesh(core_axis_name="c", subcore_axis_name="s",
                                  num_cores=NC, num_subcores=NS)
    return pl.kernel(
        _sc_scatter_body,
        out_type=(rows, rows, rows),
        mesh=mesh,
        scratch_types=(
            pltpu.VMEM((SCH,), jnp.int32),
            pltpu.VMEM((SCH,), jnp.int32),
            pltpu.VMEM((SCH, DM), jnp.float32),
            pltpu.VMEM((SCH, DM), jnp.float32),
            pltpu.SemaphoreType.DMA,
            pltpu.SemaphoreType.DMA,
            pltpu.SemaphoreType.DMA,
            pltpu.SemaphoreType.DMA,
            pltpu.SemaphoreType.DMA,
            pltpu.SemaphoreType.DMA,
        ),
    )


ABLK = 2048


def _attn_body(q_ref, k_ref, v_ref, o_ref):
    for c in range(ABLK // CHUNK):
        sl = pl.ds(c * CHUNK, CHUNK)
        qb = q_ref[sl, :]
        kb = k_ref[sl, :]
        vb = v_ref[sl, :]
        s = lax.dot_general(qb, kb, (((1,), (1,)), ((), ())),
                            preferred_element_type=jnp.float32)
        s = s / np.float32(np.sqrt(DM))
        m = jnp.max(s, axis=-1, keepdims=True)
        e = jnp.exp(s - m)
        p = e / jnp.sum(e, axis=-1, keepdims=True)
        o = lax.dot_general(p, vb, (((1,), (0,)), ((), ())),
                            preferred_element_type=jnp.float32)
        o_ref[sl, :] = o * np.float32(1.0 / NB)


def _attn(qg, kg, vg):
    spec = pl.BlockSpec((ABLK, DM), lambda i: (i, 0))
    return pl.pallas_call(
        _attn_body,
        grid=(BATCH * SEQ // ABLK,),
        in_specs=[spec, spec, spec],
        out_specs=spec,
        out_shape=jax.ShapeDtypeStruct((BATCH * SEQ, DM), jnp.float32),
    )(qg, kg, vg)


def kernel(q, k, v, projection_matrix):
    projected = jnp.matmul(q, projection_matrix)
    hashes = jnp.sign(projected)
    bucket_range = jnp.asarray([2.0 ** i for i in range(NB // 2)],
                               dtype=jnp.float32)
    bucket_ids = jnp.sum(hashes * bucket_range, axis=-1)
    bucket_ids = bucket_ids.astype(jnp.int32) % NB

    didx = _build_perm(bucket_ids.reshape(BATCH, 1, SEQ))
    didx_flat = (didx.reshape(BATCH, SEQ)
                 + jnp.arange(BATCH, dtype=jnp.int32)[:, None] * SEQ
                 ).reshape(BATCH * SEQ)

    qf = q.reshape(BATCH * SEQ, DM)
    kf = k.reshape(BATCH * SEQ, DM)
    vf = v.reshape(BATCH * SEQ, DM)
    qg, kg, vg = _sc_scatter()(qf, kf, vf, didx_flat)

    out = _attn(qg, kg, vg)
    return out.reshape(BATCH, SEQ, DM)


# speedup vs baseline: 1.0405x; 1.0405x over previous
import functools

import numpy as np
import jax
import jax.numpy as jnp
from jax import lax
from jax.experimental import pallas as pl
from jax.experimental.pallas import tpu as pltpu
from jax.experimental.pallas import tpu_sc as plsc

DM = 768
NB = 64
BATCH = 2
SEQ = 8192
CHUNK = SEQ // NB
TILE = 256
NT = SEQ // TILE
NC, NS = 2, 16
NW = NC * NS
RPW = SEQ // NW
SCH = 32
NBUF = 4


def _perm_body(buck_ref, out_ref):
    buck = buck_ref[0, 0, :]
    cid = lax.broadcasted_iota(jnp.int32, (NB, SEQ), 0)
    oh = jnp.where(cid == buck[None, :], 1.0, 0.0).astype(jnp.float32)

    r = lax.broadcasted_iota(jnp.int32, (TILE, TILE), 0)
    c = lax.broadcasted_iota(jnp.int32, (TILE, TILE), 1)
    tri = jnp.where(r <= c, 1.0, 0.0).astype(jnp.float32)

    lane_last = lax.broadcasted_iota(jnp.int32, (NB, TILE), 1) == (TILE - 1)
    run = jnp.zeros((NB, 1), jnp.float32)
    parts = []
    for t in range(NT):
        oh_t = oh[:, t * TILE:(t + 1) * TILE]
        c_t = lax.dot_general(oh_t, tri, (((1,), (0,)), ((), ())),
                              preferred_element_type=jnp.float32)
        c_t = c_t + run
        run = jnp.sum(jnp.where(lane_last, c_t, 0.0), axis=1, keepdims=True)
        parts.append(c_t)
    csum = jnp.concatenate(parts, axis=1)

    hi = jnp.floor(run * (1.0 / 32.0))
    lo = run - hi * 32.0
    rr = lax.broadcasted_iota(jnp.int32, (NB, NB), 0)
    cc = lax.broadcasted_iota(jnp.int32, (NB, NB), 1)
    lstrict = jnp.where(cc < rr, 1.0, 0.0).astype(jnp.float32)
    dn = (((1,), (0,)), ((), ()))
    starts = (lax.dot_general(lstrict, hi, dn, preferred_element_type=jnp.float32) * 32.0
              + lax.dot_general(lstrict, lo, dn, preferred_element_type=jnp.float32))

    val = oh * (csum + starts - 1.0)
    pos = jnp.sum(val, axis=0)
    out_ref[0, 0, :] = pos.astype(jnp.int32)


def _build_perm(buckets3):
    return pl.pallas_call(
        _perm_body,
        grid=(BATCH,),
        in_specs=[pl.BlockSpec((1, 1, SEQ), lambda i: (i, 0, 0))],
        out_specs=pl.BlockSpec((1, 1, SEQ), lambda i: (i, 0, 0)),
        out_shape=jax.ShapeDtypeStruct((BATCH, 1, SEQ), jnp.int32),
    )(buckets3)


def _sc_scatter_body(qf, kf, vf, didx, qg, kg, vg, *scr):
    cid = lax.axis_index("c")
    sid = lax.axis_index("s")
    base = (sid * NC + cid) * RPW
    srcs = (qf, kf, vf)
    dsts = (qg, kg, vg)
    idxb = scr[:NBUF]
    rowb = scr[NBUF:2 * NBUF]
    isem = scr[2 * NBUF:3 * NBUF]
    gsem = scr[3 * NBUF:4 * NBUF]
    ssem = scr[4 * NBUF:5 * NBUF]
    nch = RPW // SCH
    ni = 3 * BATCH * nch

    def chunk(i):
        t, r = divmod(i, BATCH * nch)
        b, ch = divmod(r, nch)
        return t, b * SEQ + base + ch * SCH

    gdesc = {}

    def prefetch(i):
        s = i % NBUF
        t, row = chunk(i)
        d1 = pltpu.async_copy(didx.at[pl.ds(row, SCH)], idxb[s], isem[s])
        d2 = pltpu.async_copy(srcs[t].at[pl.ds(row, SCH)], rowb[s], gsem[s])
        gdesc[i] = (d1, d2)

    sdesc = {}
    for i in range(NBUF - 1):
        prefetch(i)
    for i in range(ni):
        s = i % NBUF
        t, _ = chunk(i)
        d1, d2 = gdesc.pop(i)
        d1.wait()
        d2.wait()
        sdesc[i] = pltpu.async_copy(rowb[s], dsts[t].at[idxb[s]], ssem[s])
        if i + NBUF - 1 < ni:
            if i >= 1:
                sdesc.pop(i - 1).wait()
            prefetch(i + NBUF - 1)
        elif i >= 1 and (i - 1) in sdesc:
            sdesc.pop(i - 1).wait()
    for j in sorted(sdesc):
        sdesc.pop(j).wait()


@functools.cache
def _sc_scatter():
    rows = jax.ShapeDtypeStruct((BATCH * SEQ, DM), jnp.float32)
    mesh = plsc.VectorSubcoreMesh(core_axis_name="c", subcore_axis_name="s",
                                  num_cores=NC, num_subcores=NS)
    return pl.kernel(
        _sc_scatter_body,
        out_type=(rows, rows, rows),
        mesh=mesh,
        scratch_types=(
            tuple(pltpu.VMEM((SCH,), jnp.int32) for _ in range(NBUF))
            + tuple(pltpu.VMEM((SCH, DM), jnp.float32) for _ in range(NBUF))
            + tuple(pltpu.SemaphoreType.DMA for _ in range(3 * NBUF))
        ),
    )


ABLK = 1024


def _attn_body(q_ref, k_ref, v_ref, o_ref):
    for c in range(ABLK // CHUNK):
        sl = pl.ds(c * CHUNK, CHUNK)
        qb = q_ref[sl, :]
        kb = k_ref[sl, :]
        vb = v_ref[sl, :]
        s = lax.dot_general(qb, kb, (((1,), (1,)), ((), ())),
                            preferred_element_type=jnp.float32)
        s = s / np.float32(np.sqrt(DM))
        m = jnp.max(s, axis=-1, keepdims=True)
        e = jnp.exp(s - m)
        p = e / jnp.sum(e, axis=-1, keepdims=True)
        o = lax.dot_general(p, vb, (((1,), (0,)), ((), ())),
                            preferred_element_type=jnp.float32)
        o_ref[sl, :] = o * np.float32(1.0 / NB)


def _attn(qg, kg, vg):
    spec = pl.BlockSpec((ABLK, DM), lambda i: (i, 0))
    return pl.pallas_call(
        _attn_body,
        grid=(BATCH * SEQ // ABLK,),
        in_specs=[spec, spec, spec],
        out_specs=spec,
        out_shape=jax.ShapeDtypeStruct((BATCH * SEQ, DM), jnp.float32),
    )(qg, kg, vg)


def kernel(q, k, v, projection_matrix):
    projected = jnp.matmul(q, projection_matrix)
    hashes = jnp.sign(projected)
    bucket_range = jnp.asarray([2.0 ** i for i in range(NB // 2)],
                               dtype=jnp.float32)
    bucket_ids = jnp.sum(hashes * bucket_range, axis=-1)
    bucket_ids = bucket_ids.astype(jnp.int32) % NB

    didx = _build_perm(bucket_ids.reshape(BATCH, 1, SEQ))
    didx_flat = (didx.reshape(BATCH, SEQ)
                 + jnp.arange(BATCH, dtype=jnp.int32)[:, None] * SEQ
                 ).reshape(BATCH * SEQ)

    qf = q.reshape(BATCH * SEQ, DM)
    kf = k.reshape(BATCH * SEQ, DM)
    vf = v.reshape(BATCH * SEQ, DM)
    qg, kg, vg = _sc_scatter()(qf, kf, vf, didx_flat)

    out = _attn(qg, kg, vg)
    return out.reshape(BATCH, SEQ, DM)


# speedup vs baseline: 1.0434x; 1.0028x over previous
import functools

import numpy as np
import jax
import jax.numpy as jnp
from jax import lax
from jax.experimental import pallas as pl
from jax.experimental.pallas import tpu as pltpu
from jax.experimental.pallas import tpu_sc as plsc

DM = 768
NB = 64
BATCH = 2
SEQ = 8192
CHUNK = SEQ // NB
TILE = 256
NT = SEQ // TILE
NC, NS = 2, 16
NW = NC * NS
RPW = SEQ // NW
SCH = 32
NBUF = 5


def _perm_body(buck_ref, out_ref):
    buck = buck_ref[0, 0, :]
    cid = lax.broadcasted_iota(jnp.int32, (NB, SEQ), 0)
    oh = jnp.where(cid == buck[None, :], 1.0, 0.0).astype(jnp.float32)

    r = lax.broadcasted_iota(jnp.int32, (TILE, TILE), 0)
    c = lax.broadcasted_iota(jnp.int32, (TILE, TILE), 1)
    tri = jnp.where(r <= c, 1.0, 0.0).astype(jnp.float32)

    lane_last = lax.broadcasted_iota(jnp.int32, (NB, TILE), 1) == (TILE - 1)
    run = jnp.zeros((NB, 1), jnp.float32)
    parts = []
    for t in range(NT):
        oh_t = oh[:, t * TILE:(t + 1) * TILE]
        c_t = lax.dot_general(oh_t, tri, (((1,), (0,)), ((), ())),
                              preferred_element_type=jnp.float32)
        c_t = c_t + run
        run = jnp.sum(jnp.where(lane_last, c_t, 0.0), axis=1, keepdims=True)
        parts.append(c_t)
    csum = jnp.concatenate(parts, axis=1)

    hi = jnp.floor(run * (1.0 / 32.0))
    lo = run - hi * 32.0
    rr = lax.broadcasted_iota(jnp.int32, (NB, NB), 0)
    cc = lax.broadcasted_iota(jnp.int32, (NB, NB), 1)
    lstrict = jnp.where(cc < rr, 1.0, 0.0).astype(jnp.float32)
    dn = (((1,), (0,)), ((), ()))
    starts = (lax.dot_general(lstrict, hi, dn, preferred_element_type=jnp.float32) * 32.0
              + lax.dot_general(lstrict, lo, dn, preferred_element_type=jnp.float32))

    val = oh * (csum + starts - 1.0)
    pos = jnp.sum(val, axis=0)
    out_ref[0, 0, :] = pos.astype(jnp.int32)


def _build_perm(buckets3):
    return pl.pallas_call(
        _perm_body,
        grid=(BATCH,),
        in_specs=[pl.BlockSpec((1, 1, SEQ), lambda i: (i, 0, 0))],
        out_specs=pl.BlockSpec((1, 1, SEQ), lambda i: (i, 0, 0)),
        out_shape=jax.ShapeDtypeStruct((BATCH, 1, SEQ), jnp.int32),
    )(buckets3)


def _sc_scatter_body(qf, kf, vf, didx, qg, kg, vg, *scr):
    cid = lax.axis_index("c")
    sid = lax.axis_index("s")
    base = (sid * NC + cid) * RPW
    srcs = (qf, kf, vf)
    dsts = (qg, kg, vg)
    idxb = scr[:NBUF]
    rowb = scr[NBUF:2 * NBUF]
    isem = scr[2 * NBUF:3 * NBUF]
    gsem = scr[3 * NBUF:4 * NBUF]
    ssem = scr[4 * NBUF:5 * NBUF]
    nch = RPW // SCH
    ni = 3 * BATCH * nch

    def chunk(i):
        t, r = divmod(i, BATCH * nch)
        b, ch = divmod(r, nch)
        return t, b * SEQ + base + ch * SCH

    gdesc = {}

    def prefetch(i):
        s = i % NBUF
        t, row = chunk(i)
        d1 = pltpu.async_copy(didx.at[pl.ds(row, SCH)], idxb[s], isem[s])
        d2 = pltpu.async_copy(srcs[t].at[pl.ds(row, SCH)], rowb[s], gsem[s])
        gdesc[i] = (d1, d2)

    sdesc = {}
    for i in range(NBUF - 1):
        prefetch(i)
    for i in range(ni):
        s = i % NBUF
        t, _ = chunk(i)
        d1, d2 = gdesc.pop(i)
        d1.wait()
        d2.wait()
        sdesc[i] = pltpu.async_copy(rowb[s], dsts[t].at[idxb[s]], ssem[s])
        if i + NBUF - 1 < ni:
            if i >= 1:
                sdesc.pop(i - 1).wait()
            prefetch(i + NBUF - 1)
        elif i >= 1 and (i - 1) in sdesc:
            sdesc.pop(i - 1).wait()
    for j in sorted(sdesc):
        sdesc.pop(j).wait()


@functools.cache
def _sc_scatter():
    rows = jax.ShapeDtypeStruct((BATCH * SEQ, DM), jnp.float32)
    mesh = plsc.VectorSubcoreMesh(core_axis_name="c", subcore_axis_name="s",
                                  num_cores=NC, num_subcores=NS)
    return pl.kernel(
        _sc_scatter_body,
        out_type=(rows, rows, rows),
        mesh=mesh,
        scratch_types=(
            tuple(pltpu.VMEM((SCH,), jnp.int32) for _ in range(NBUF))
            + tuple(pltpu.VMEM((SCH, DM), jnp.float32) for _ in range(NBUF))
            + tuple(pltpu.SemaphoreType.DMA for _ in range(3 * NBUF))
        ),
    )


ABLK = 1024


def _attn_body(q_ref, k_ref, v_ref, o_ref):
    for c in range(ABLK // CHUNK):
        sl = pl.ds(c * CHUNK, CHUNK)
        qb = q_ref[sl, :]
        kb = k_ref[sl, :]
        vb = v_ref[sl, :]
        s = lax.dot_general(qb, kb, (((1,), (1,)), ((), ())),
                            preferred_element_type=jnp.float32)
        s = s / np.float32(np.sqrt(DM))
        m = jnp.max(s, axis=-1, keepdims=True)
        e = jnp.exp(s - m)
        p = e / jnp.sum(e, axis=-1, keepdims=True)
        o = lax.dot_general(p, vb, (((1,), (0,)), ((), ())),
                            preferred_element_type=jnp.float32)
        o_ref[sl, :] = o * np.float32(1.0 / NB)


def _attn(qg, kg, vg):
    spec = pl.BlockSpec((ABLK, DM), lambda i: (i, 0))
    return pl.pallas_call(
        _attn_body,
        grid=(BATCH * SEQ // ABLK,),
        in_specs=[spec, spec, spec],
        out_specs=spec,
        out_shape=jax.ShapeDtypeStruct((BATCH * SEQ, DM), jnp.float32),
    )(qg, kg, vg)


def kernel(q, k, v, projection_matrix):
    projected = jnp.matmul(q, projection_matrix)
    hashes = jnp.sign(projected)
    bucket_range = jnp.asarray([2.0 ** i for i in range(NB // 2)],
                               dtype=jnp.float32)
    bucket_ids = jnp.sum(hashes * bucket_range, axis=-1)
    bucket_ids = bucket_ids.astype(jnp.int32) % NB

    didx = _build_perm(bucket_ids.reshape(BATCH, 1, SEQ))
    didx_flat = (didx.reshape(BATCH, SEQ)
                 + jnp.arange(BATCH, dtype=jnp.int32)[:, None] * SEQ
                 ).reshape(BATCH * SEQ)

    qf = q.reshape(BATCH * SEQ, DM)
    kf = k.reshape(BATCH * SEQ, DM)
    vf = v.reshape(BATCH * SEQ, DM)
    qg, kg, vg = _sc_scatter()(qf, kf, vf, didx_flat)

    out = _attn(qg, kg, vg)
    return out.reshape(BATCH, SEQ, DM)
